# baseline (speedup 1.0000x reference)
import jax
import jax.numpy as jnp
from jax import lax
from jax.experimental import pallas as pl
from jax.experimental.pallas import tpu as pltpu
from jax.experimental.pallas import tpu_sc as plsc

N_NODES = 10000
N_PAD = 10240
SINK = N_NODES
E_EDGES = 320000
E_PAD = 327680
IDX_ROWS = E_PAD // 128
E_HALF = E_PAD // 2
HALF_ROWS = IDX_ROWS // 2
ROWS_PER_TILE = HALF_ROWS // 16
SC_ROWS_PER_TILE = HALF_ROWS // 32
D = 128
HID = 64
OUT = 128
G = 64
MLP_BLK = 2048


def _proj_body(p0_ref, p1_ref, p2_ref, p3_ref, wcat_ref, ab_ref):
    state = (p0_ref[...] + p1_ref[...]) + (p2_ref[...] + p3_ref[...])
    ab_ref[...] = jnp.dot(state, wcat_ref[...],
                          preferred_element_type=jnp.float32)


def _proj(p0, p1, p2, p3, wcat):
    blk = 2048
    grid = N_PAD // blk
    p_spec = pl.BlockSpec((blk, D), lambda i: (i, 0))
    return pl.pallas_call(
        _proj_body,
        grid=(grid,),
        in_specs=[p_spec, p_spec, p_spec, p_spec,
                  pl.BlockSpec((D, D), lambda i: (0, 0))],
        out_specs=pl.BlockSpec((blk, D), lambda i: (i, 0)),
        out_shape=jax.ShapeDtypeStruct((N_PAD, D), jnp.float32),
    )(p0, p1, p2, p3, wcat)


GATHER_GROUP = 1
G_GROUPS = ROWS_PER_TILE // GATHER_GROUP


def _gather_sc_body(ab_hbm, src_hbm, dst_hbm, pre_s_hbm, pre_d_hbm,
                    idx0_v, idx1_v, rows0_v, rows1_v, ab_sp,
                    gsem, osem0, osem1, isem0, isem1):
    c = lax.axis_index("c")
    s = lax.axis_index("s")
    bufs = [(idx0_v, rows0_v, osem0, isem0), (idx1_v, rows1_v, osem1, isem1)]
    nrows = GATHER_GROUP * 128

    pltpu.sync_copy(ab_hbm.at[pl.ds(s * (N_PAD // 16), N_PAD // 16)],
                    ab_sp.at[pl.ds(s * (N_PAD // 16), N_PAD // 16)])
    plsc.subcore_barrier()

    def run(idx2d, out):
        tile_row = s * ROWS_PER_TILE

        def idx_dma(g, b):
            idx_v = bufs[b][0]
            isem = bufs[b][3]
            return pltpu.make_async_copy(
                idx2d.at[pl.ds(tile_row + g * GATHER_GROUP, GATHER_GROUP)],
                idx_v, isem)

        def out_dma(g, b):
            rows_v = bufs[b][1]
            osem = bufs[b][2]
            return pltpu.make_async_copy(
                rows_v,
                out.at[pl.ds((tile_row + g * GATHER_GROUP) * 128, nrows)],
                osem)

        idx_dma(0, 0).start()
        idx_dma(1, 1).start()

        def group(g, _):
            b = lax.rem(g, 2)

            def body(bi):
                idx_v, rows_v, _, _ = bufs[bi]
                @pl.when(g >= 2)
                def _():
                    out_dma(g, bi).wait()
                idx_dma(g, bi).wait()
                descs = [
                    pltpu.make_async_copy(
                        ab_sp.at[idx_v.at[j]],
                        rows_v.at[pl.ds(j * 128, 128)],
                        gsem,
                    )
                    for j in range(GATHER_GROUP)
                ]
                for d_ in descs:
                    d_.start()
                for d_ in descs:
                    d_.wait()
                out_dma(g, bi).start()
                @pl.when(g + 2 < G_GROUPS)
                def _():
                    idx_dma(g + 2, bi).start()

            @pl.when(b == 0)
            def _():
                body(0)

            @pl.when(b == 1)
            def _():
                body(1)

            return 0

        lax.fori_loop(0, G_GROUPS, group, 0)
        out_dma(G_GROUPS - 2, 0).wait()
        out_dma(G_GROUPS - 1, 1).wait()

    @pl.when(c == 0)
    def _():
        run(src_hbm, pre_s_hbm)

    @pl.when(c == 1)
    def _():
        run(dst_hbm, pre_d_hbm)


def _gather(ab, src2d, dst2d):
    return pl.kernel(
        _gather_sc_body,
        out_type=[
            jax.ShapeDtypeStruct((E_HALF, D), jnp.float32),
            jax.ShapeDtypeStruct((E_HALF, D), jnp.float32),
        ],
        mesh=plsc.VectorSubcoreMesh(core_axis_name="c", subcore_axis_name="s"),
        scratch_types=[
            pltpu.VMEM((GATHER_GROUP, 128), jnp.int32),
            pltpu.VMEM((GATHER_GROUP, 128), jnp.int32),
            pltpu.VMEM((GATHER_GROUP * 128, D), jnp.float32),
            pltpu.VMEM((GATHER_GROUP * 128, D), jnp.float32),
            pltpu.VMEM_SHARED((N_PAD, D), jnp.float32),
            pltpu.SemaphoreType.DMA,
            pltpu.SemaphoreType.DMA,
            pltpu.SemaphoreType.DMA,
            pltpu.SemaphoreType.DMA,
            pltpu.SemaphoreType.DMA,
        ],
    )(ab, src2d, dst2d)


def _mlp_body(pre_s_ref, pre_d_ref, bin_ref, wh_ref, bh_ref, wout_ref,
              bout_ref, m_ref):
    pre = (pre_s_ref[:, :HID].astype(jnp.float32)
           + pre_d_ref[:, HID:].astype(jnp.float32))
    h = jax.nn.relu(pre + bin_ref[...])
    h = jax.nn.relu(
        jnp.dot(h, wh_ref[...], preferred_element_type=jnp.float32)
        + bh_ref[...])
    m_ref[...] = jax.nn.relu(
        jnp.dot(h, wout_ref[...], preferred_element_type=jnp.float32)
        + bout_ref[...])


def _mlp(pre_s, pre_d, bin_s, wh, bh, wout, bout):
    grid = E_HALF // MLP_BLK
    return pl.pallas_call(
        _mlp_body,
        grid=(grid,),
        in_specs=[
            pl.BlockSpec((MLP_BLK, D), lambda i: (i, 0)),
            pl.BlockSpec((MLP_BLK, D), lambda i: (i, 0)),
            pl.BlockSpec((1, HID), lambda i: (0, 0)),
            pl.BlockSpec((HID, HID), lambda i: (0, 0)),
            pl.BlockSpec((1, HID), lambda i: (0, 0)),
            pl.BlockSpec((HID, D), lambda i: (0, 0)),
            pl.BlockSpec((1, D), lambda i: (0, 0)),
        ],
        out_specs=pl.BlockSpec((MLP_BLK, D), lambda i: (i, 0)),
        out_shape=jax.ShapeDtypeStruct((E_HALF, D), jnp.float32),
    )(pre_s, pre_d, bin_s, wh, bh, wout, bout)


SC_ROWS_PER_TILE = IDX_ROWS // 32
def _scatter_sc_body(m_hbm, dst_hbm, zeros_hbm, p0_hbm, p1_hbm,
                     idx0_v, idx1_v, rows0_v, rows1_v, acc,
                     isem0, isem1, msem0, msem1):
    c = lax.axis_index("c")
    s = lax.axis_index("s")
    bufs = [(idx0_v, rows0_v, isem0, msem0), (idx1_v, rows1_v, isem1, msem1)]

    def run(out):
        tile_row = (c * 16 + s) * SC_ROWS_PER_TILE

        def idx_dma(g, b):
            return pltpu.make_async_copy(
                dst_hbm.at[pl.ds(tile_row + g, 1)], bufs[b][0], bufs[b][2])

        def m_dma(g, b):
            return pltpu.make_async_copy(
                m_hbm.at[pl.ds((tile_row + g) * 128, 128)],
                bufs[b][1], bufs[b][3])

        idx_dma(0, 0).start()
        m_dma(0, 0).start()
        idx_dma(1, 1).start()
        m_dma(1, 1).start()
        pltpu.sync_copy(zeros_hbm, acc.at[pl.ds(s * 640, 640)])
        plsc.subcore_barrier()

        def group(g, _):
            b = lax.rem(g, 2)

            def body(bi):
                idx_v, rows_v, _, _ = bufs[bi]
                idx_dma(g, bi).wait()
                m_dma(g, bi).wait()
                pltpu.sync_copy(rows_v, acc.at[idx_v.at[0]], add=True)
                @pl.when(g + 2 < SC_ROWS_PER_TILE)
                def _():
                    idx_dma(g + 2, bi).start()
                    m_dma(g + 2, bi).start()

            @pl.when(b == 0)
            def _():
                body(0)

            @pl.when(b == 1)
            def _():
                body(1)

            return 0

        lax.fori_loop(0, SC_ROWS_PER_TILE, group, 0)
        plsc.subcore_barrier()
        pltpu.sync_copy(acc.at[pl.ds(s * 640, 640)],
                        out.at[pl.ds(s * 640, 640)])

    @pl.when(c == 0)
    def _():
        run(p0_hbm)

    @pl.when(c == 1)
    def _():
        run(p1_hbm)


def _scatter(m, dst2d, zeros_rows):
    return pl.kernel(
        _scatter_sc_body,
        out_type=[
            jax.ShapeDtypeStruct((N_PAD, D), jnp.float32),
            jax.ShapeDtypeStruct((N_PAD, D), jnp.float32),
        ],
        mesh=plsc.VectorSubcoreMesh(core_axis_name="c", subcore_axis_name="s"),
        scratch_types=[
            pltpu.VMEM((1, 128), jnp.int32),
            pltpu.VMEM((1, 128), jnp.int32),
            pltpu.VMEM((128, D), jnp.float32),
            pltpu.VMEM((128, D), jnp.float32),
            pltpu.VMEM_SHARED((N_PAD, D), jnp.float32),
            pltpu.SemaphoreType.DMA,
            pltpu.SemaphoreType.DMA,
            pltpu.SemaphoreType.DMA,
            pltpu.SemaphoreType.DMA,
        ],
    )(m, dst2d, zeros_rows)


def _readout_body(p0_ref, p1_ref, p2_ref, p3_ref, ids_ref, w1_ref, b1_ref,
                  w2_ref, b2_ref, wo_ref, bo_ref, out_ref):
    gids = lax.broadcasted_iota(jnp.int32, (G, N_PAD), 0).astype(jnp.float32)
    oh = jnp.where(ids_ref[...] == gids, 1.0, 0.0)
    state = (p0_ref[...] + p1_ref[...]) + (p2_ref[...] + p3_ref[...])
    mol = jnp.dot(oh, state, preferred_element_type=jnp.float32)
    h = jax.nn.relu(
        jnp.dot(mol, w1_ref[...], preferred_element_type=jnp.float32)
        + b1_ref[...])
    h = jax.nn.relu(
        jnp.dot(h, w2_ref[...], preferred_element_type=jnp.float32)
        + b2_ref[...])
    out_ref[...] = (
        jnp.dot(h, wo_ref[...], preferred_element_type=jnp.float32)
        + bo_ref[...])


def _readout(p0, p1, p2, p3, ids_f, w1, b1, w2, b2, wo, bo):
    return pl.pallas_call(
        _readout_body,
        out_shape=jax.ShapeDtypeStruct((G, OUT), jnp.float32),
    )(p0, p1, p2, p3, ids_f, w1, b1, w2, b2, wo, bo)


def kernel(x, edge_index, graph_ids, Win, bin_, Wh, bh, Wout, bout,
           W1, b1, W2, b2, Wo, bo):
    src = edge_index[0]
    dst = edge_index[1]
    pad_e = E_PAD - E_EDGES
    src2d = jnp.concatenate(
        [src, jnp.zeros((pad_e,), jnp.int32)]).reshape(IDX_ROWS, 128)
    dst2d = jnp.concatenate(
        [dst, jnp.full((pad_e,), SINK, jnp.int32)]).reshape(IDX_ROWS, 128)

    xp = jnp.pad(x, ((0, N_PAD - N_NODES), (0, 0)))
    ids_f = jnp.concatenate(
        [graph_ids, jnp.full((N_PAD - N_NODES,), G, jnp.int32)]
    ).astype(jnp.float32).reshape(1, N_PAD)
    zeros_rows = jnp.zeros((640, D), jnp.float32)
    zeros_n = jnp.zeros((N_PAD, D), jnp.float32)

    src_h = (src2d[:HALF_ROWS], src2d[HALF_ROWS:])
    dst_h = (dst2d[:HALF_ROWS], dst2d[HALF_ROWS:])

    partials = [xp, zeros_n, zeros_n, zeros_n]
    for step in range(3):
        wcat = jnp.concatenate([Win[step][:D], Win[step][D:]], axis=1)
        ab = _proj(*partials, wcat)
        new_partials = []
        for ch in range(2):
            pre_s, pre_d = _gather(ab, src_h[ch], dst_h[ch])
            m = _mlp(pre_s, pre_d, bin_[step].reshape(1, HID), Wh[step],
                     bh[step].reshape(1, HID), Wout[step],
                     bout[step].reshape(1, D))
            pa, pb = _scatter(m, dst_h[ch], zeros_rows)
            new_partials += [pa, pb]
        partials = new_partials

    return _readout(*partials, ids_f, W1, b1.reshape(1, HID), W2,
                    b2.reshape(1, HID), Wo, bo.reshape(1, OUT))


# speedup vs baseline: 1.1496x; 1.1496x over previous
import jax
import jax.numpy as jnp
from jax import lax
from jax.experimental import pallas as pl
from jax.experimental.pallas import tpu as pltpu
from jax.experimental.pallas import tpu_sc as plsc

N_NODES = 10000
N_PAD = 10240
SINK = N_NODES
E_EDGES = 320000
E_PAD = 327680
IDX_ROWS = E_PAD // 128
ROWS_PER_TILE = IDX_ROWS // 16
GROUPS_PER_TILE = ROWS_PER_TILE // 8
D = 128
HID = 64
OUT = 128
G = 64
MLP_BLK = 4096


def _proj_body(p0_ref, p1_ref, wcat_ref, ab_ref):
    state = p0_ref[...] + p1_ref[...]
    ab_ref[...] = jnp.dot(state, wcat_ref[...],
                          preferred_element_type=jnp.float32)


def _proj(p0, p1, wcat):
    blk = 2048
    grid = N_PAD // blk
    return pl.pallas_call(
        _proj_body,
        grid=(grid,),
        in_specs=[
            pl.BlockSpec((blk, D), lambda i: (i, 0)),
            pl.BlockSpec((blk, D), lambda i: (i, 0)),
            pl.BlockSpec((D, D), lambda i: (0, 0)),
        ],
        out_specs=pl.BlockSpec((blk, D), lambda i: (i, 0)),
        out_shape=jax.ShapeDtypeStruct((N_PAD, D), jnp.float32),
    )(p0, p1, wcat)


GATHER_GROUP = 1
G_GROUPS = ROWS_PER_TILE // GATHER_GROUP


def _gather_sc_body(ab_hbm, src_hbm, dst_hbm, pre_s_hbm, pre_d_hbm,
                    idx0_v, idx1_v, rows0_v, rows1_v, ab_sp,
                    gsem, osem0, osem1, isem0, isem1):
    c = lax.axis_index("c")
    s = lax.axis_index("s")
    bufs = [(idx0_v, rows0_v, osem0, isem0), (idx1_v, rows1_v, osem1, isem1)]
    nrows = GATHER_GROUP * 128

    pltpu.sync_copy(ab_hbm.at[pl.ds(s * (N_PAD // 16), N_PAD // 16)],
                    ab_sp.at[pl.ds(s * (N_PAD // 16), N_PAD // 16)])
    plsc.subcore_barrier()

    def run(idx2d, out):
        tile_row = s * ROWS_PER_TILE

        def idx_dma(g, b):
            idx_v = bufs[b][0]
            isem = bufs[b][3]
            return pltpu.make_async_copy(
                idx2d.at[pl.ds(tile_row + g * GATHER_GROUP, GATHER_GROUP)],
                idx_v, isem)

        def out_dma(g, b):
            rows_v = bufs[b][1]
            osem = bufs[b][2]
            return pltpu.make_async_copy(
                rows_v,
                out.at[pl.ds((tile_row + g * GATHER_GROUP) * 128, nrows)],
                osem)

        idx_dma(0, 0).start()
        idx_dma(1, 1).start()

        def group(g, _):
            b = lax.rem(g, 2)

            def body(bi):
                idx_v, rows_v, _, _ = bufs[bi]
                @pl.when(g >= 2)
                def _():
                    out_dma(g, bi).wait()
                idx_dma(g, bi).wait()
                descs = [
                    pltpu.make_async_copy(
                        ab_sp.at[idx_v.at[j]],
                        rows_v.at[pl.ds(j * 128, 128)],
                        gsem,
                    )
                    for j in range(GATHER_GROUP)
                ]
                for d_ in descs:
                    d_.start()
                for d_ in descs:
                    d_.wait()
                out_dma(g, bi).start()
                @pl.when(g + 2 < G_GROUPS)
                def _():
                    idx_dma(g + 2, bi).start()

            @pl.when(b == 0)
            def _():
                body(0)

            @pl.when(b == 1)
            def _():
                body(1)

            return 0

        lax.fori_loop(0, G_GROUPS, group, 0)
        out_dma(G_GROUPS - 2, 0).wait()
        out_dma(G_GROUPS - 1, 1).wait()

    @pl.when(c == 0)
    def _():
        run(src_hbm, pre_s_hbm)

    @pl.when(c == 1)
    def _():
        run(dst_hbm, pre_d_hbm)


def _gather(ab, src2d, dst2d):
    return pl.kernel(
        _gather_sc_body,
        out_type=[
            jax.ShapeDtypeStruct((E_PAD, D), jnp.float32),
            jax.ShapeDtypeStruct((E_PAD, D), jnp.float32),
        ],
        mesh=plsc.VectorSubcoreMesh(core_axis_name="c", subcore_axis_name="s"),
        scratch_types=[
            pltpu.VMEM((GATHER_GROUP, 128), jnp.int32),
            pltpu.VMEM((GATHER_GROUP, 128), jnp.int32),
            pltpu.VMEM((GATHER_GROUP * 128, D), jnp.float32),
            pltpu.VMEM((GATHER_GROUP * 128, D), jnp.float32),
            pltpu.VMEM_SHARED((N_PAD, D), jnp.float32),
            pltpu.SemaphoreType.DMA,
            pltpu.SemaphoreType.DMA,
            pltpu.SemaphoreType.DMA,
            pltpu.SemaphoreType.DMA,
            pltpu.SemaphoreType.DMA,
        ],
    )(ab, src2d, dst2d)


def _mlp_body(pre_s_ref, pre_d_ref, bin_ref, wh_ref, bh_ref, wout_ref,
              bout_ref, m_ref):
    pre = (pre_s_ref[:, :HID].astype(jnp.float32)
           + pre_d_ref[:, HID:].astype(jnp.float32))
    h = jax.nn.relu(pre + bin_ref[...])
    h = jax.nn.relu(
        jnp.dot(h, wh_ref[...], preferred_element_type=jnp.float32)
        + bh_ref[...])
    m_ref[...] = jax.nn.relu(
        jnp.dot(h, wout_ref[...], preferred_element_type=jnp.float32)
        + bout_ref[...])


def _mlp(pre_s, pre_d, bin_s, wh, bh, wout, bout):
    grid = E_PAD // MLP_BLK
    return pl.pallas_call(
        _mlp_body,
        grid=(grid,),
        in_specs=[
            pl.BlockSpec((MLP_BLK, D), lambda i: (i, 0)),
            pl.BlockSpec((MLP_BLK, D), lambda i: (i, 0)),
            pl.BlockSpec((1, HID), lambda i: (0, 0)),
            pl.BlockSpec((HID, HID), lambda i: (0, 0)),
            pl.BlockSpec((1, HID), lambda i: (0, 0)),
            pl.BlockSpec((HID, D), lambda i: (0, 0)),
            pl.BlockSpec((1, D), lambda i: (0, 0)),
        ],
        out_specs=pl.BlockSpec((MLP_BLK, D), lambda i: (i, 0)),
        out_shape=jax.ShapeDtypeStruct((E_PAD, D), jnp.float32),
    )(pre_s, pre_d, bin_s, wh, bh, wout, bout)


SC_ROWS_PER_TILE = IDX_ROWS // 32
def _scatter_sc_body(m_hbm, dst_hbm, zeros_hbm, p0_hbm, p1_hbm,
                     idx0_v, idx1_v, rows0_v, rows1_v, acc,
                     isem0, isem1, msem0, msem1):
    c = lax.axis_index("c")
    s = lax.axis_index("s")
    bufs = [(idx0_v, rows0_v, isem0, msem0), (idx1_v, rows1_v, isem1, msem1)]

    def run(out):
        tile_row = (c * 16 + s) * SC_ROWS_PER_TILE

        def idx_dma(g, b):
            return pltpu.make_async_copy(
                dst_hbm.at[pl.ds(tile_row + g, 1)], bufs[b][0], bufs[b][2])

        def m_dma(g, b):
            return pltpu.make_async_copy(
                m_hbm.at[pl.ds((tile_row + g) * 128, 128)],
                bufs[b][1], bufs[b][3])

        idx_dma(0, 0).start()
        m_dma(0, 0).start()
        idx_dma(1, 1).start()
        m_dma(1, 1).start()
        pltpu.sync_copy(zeros_hbm, acc.at[pl.ds(s * 640, 640)])
        plsc.subcore_barrier()

        def group(g, _):
            b = lax.rem(g, 2)

            def body(bi):
                idx_v, rows_v, _, _ = bufs[bi]
                idx_dma(g, bi).wait()
                m_dma(g, bi).wait()
                pltpu.sync_copy(rows_v, acc.at[idx_v.at[0]], add=True)
                @pl.when(g + 2 < SC_ROWS_PER_TILE)
                def _():
                    idx_dma(g + 2, bi).start()
                    m_dma(g + 2, bi).start()

            @pl.when(b == 0)
            def _():
                body(0)

            @pl.when(b == 1)
            def _():
                body(1)

            return 0

        lax.fori_loop(0, SC_ROWS_PER_TILE, group, 0)
        plsc.subcore_barrier()
        pltpu.sync_copy(acc.at[pl.ds(s * 640, 640)],
                        out.at[pl.ds(s * 640, 640)])

    @pl.when(c == 0)
    def _():
        run(p0_hbm)

    @pl.when(c == 1)
    def _():
        run(p1_hbm)


def _scatter(m, dst2d, zeros_rows):
    return pl.kernel(
        _scatter_sc_body,
        out_type=[
            jax.ShapeDtypeStruct((N_PAD, D), jnp.float32),
            jax.ShapeDtypeStruct((N_PAD, D), jnp.float32),
        ],
        mesh=plsc.VectorSubcoreMesh(core_axis_name="c", subcore_axis_name="s"),
        scratch_types=[
            pltpu.VMEM((1, 128), jnp.int32),
            pltpu.VMEM((1, 128), jnp.int32),
            pltpu.VMEM((128, D), jnp.float32),
            pltpu.VMEM((128, D), jnp.float32),
            pltpu.VMEM_SHARED((N_PAD, D), jnp.float32),
            pltpu.SemaphoreType.DMA,
            pltpu.SemaphoreType.DMA,
            pltpu.SemaphoreType.DMA,
            pltpu.SemaphoreType.DMA,
        ],
    )(m, dst2d, zeros_rows)


def _readout_body(p0_ref, p1_ref, ids_ref, w1_ref, b1_ref, w2_ref, b2_ref,
                  wo_ref, bo_ref, out_ref):
    gids = lax.broadcasted_iota(jnp.int32, (G, N_PAD), 0).astype(jnp.float32)
    oh = jnp.where(ids_ref[...] == gids, 1.0, 0.0)
    state = p0_ref[...] + p1_ref[...]
    mol = jnp.dot(oh, state, preferred_element_type=jnp.float32)
    h = jax.nn.relu(
        jnp.dot(mol, w1_ref[...], preferred_element_type=jnp.float32)
        + b1_ref[...])
    h = jax.nn.relu(
        jnp.dot(h, w2_ref[...], preferred_element_type=jnp.float32)
        + b2_ref[...])
    out_ref[...] = (
        jnp.dot(h, wo_ref[...], preferred_element_type=jnp.float32)
        + bo_ref[...])


def _readout(p0, p1, ids_f, w1, b1, w2, b2, wo, bo):
    return pl.pallas_call(
        _readout_body,
        out_shape=jax.ShapeDtypeStruct((G, OUT), jnp.float32),
    )(p0, p1, ids_f, w1, b1, w2, b2, wo, bo)


def kernel(x, edge_index, graph_ids, Win, bin_, Wh, bh, Wout, bout,
           W1, b1, W2, b2, Wo, bo):
    src = edge_index[0]
    dst = edge_index[1]
    pad_e = E_PAD - E_EDGES
    src2d = jnp.concatenate(
        [src, jnp.zeros((pad_e,), jnp.int32)]).reshape(IDX_ROWS, 128)
    dst2d = jnp.concatenate(
        [dst, jnp.full((pad_e,), SINK, jnp.int32)]).reshape(IDX_ROWS, 128)

    xp = jnp.pad(x, ((0, N_PAD - N_NODES), (0, 0)))
    ids_f = jnp.concatenate(
        [graph_ids, jnp.full((N_PAD - N_NODES,), G, jnp.int32)]
    ).astype(jnp.float32).reshape(1, N_PAD)
    zeros_rows = jnp.zeros((640, D), jnp.float32)
    zeros_n = jnp.zeros((N_PAD, D), jnp.float32)

    p0, p1 = xp, zeros_n
    for step in range(3):
        wcat = jnp.concatenate([Win[step][:D], Win[step][D:]], axis=1)
        ab = _proj(p0, p1, wcat)
        pre_s, pre_d = _gather(ab, src2d, dst2d)
        m = _mlp(pre_s, pre_d, bin_[step].reshape(1, HID), Wh[step],
                 bh[step].reshape(1, HID), Wout[step],
                 bout[step].reshape(1, D))
        p0, p1 = _scatter(m, dst2d, zeros_rows)

    return _readout(p0, p1, ids_f, W1, b1.reshape(1, HID), W2,
                    b2.reshape(1, HID), Wo, bo.reshape(1, OUT))


# speedup vs baseline: 1.1698x; 1.0175x over previous
import jax
import jax.numpy as jnp
from jax import lax
from jax.experimental import pallas as pl
from jax.experimental.pallas import tpu as pltpu
from jax.experimental.pallas import tpu_sc as plsc

N_NODES = 10000
N_PAD = 10240
SINK = N_NODES
E_EDGES = 320000
E_PAD = 327680
IDX_ROWS = E_PAD // 128
ROWS_PER_TILE = IDX_ROWS // 16
GROUPS_PER_TILE = ROWS_PER_TILE // 8
D = 128
HID = 64
OUT = 128
G = 64
MLP_BLK = 8192


def _proj_body(p0_ref, p1_ref, wcat_ref, ab_ref):
    state = p0_ref[...] + p1_ref[...]
    ab_ref[...] = jnp.dot(state, wcat_ref[...],
                          preferred_element_type=jnp.float32)


def _proj(p0, p1, wcat):
    blk = 2048
    grid = N_PAD // blk
    return pl.pallas_call(
        _proj_body,
        grid=(grid,),
        in_specs=[
            pl.BlockSpec((blk, D), lambda i: (i, 0)),
            pl.BlockSpec((blk, D), lambda i: (i, 0)),
            pl.BlockSpec((D, D), lambda i: (0, 0)),
        ],
        out_specs=pl.BlockSpec((blk, D), lambda i: (i, 0)),
        out_shape=jax.ShapeDtypeStruct((N_PAD, D), jnp.float32),
    )(p0, p1, wcat)


GATHER_GROUP = 1
G_GROUPS = ROWS_PER_TILE // GATHER_GROUP


def _gather_sc_body(ab_hbm, src_hbm, dst_hbm, pre_s_hbm, pre_d_hbm,
                    idx0_v, idx1_v, rows0_v, rows1_v, ab_sp,
                    gsem, osem0, osem1, isem0, isem1):
    c = lax.axis_index("c")
    s = lax.axis_index("s")
    bufs = [(idx0_v, rows0_v, osem0, isem0), (idx1_v, rows1_v, osem1, isem1)]
    nrows = GATHER_GROUP * 128

    pltpu.sync_copy(ab_hbm.at[pl.ds(s * (N_PAD // 16), N_PAD // 16)],
                    ab_sp.at[pl.ds(s * (N_PAD // 16), N_PAD // 16)])
    plsc.subcore_barrier()

    def run(idx2d, out):
        tile_row = s * ROWS_PER_TILE

        def idx_dma(g, b):
            idx_v = bufs[b][0]
            isem = bufs[b][3]
            return pltpu.make_async_copy(
                idx2d.at[pl.ds(tile_row + g * GATHER_GROUP, GATHER_GROUP)],
                idx_v, isem)

        def out_dma(g, b):
            rows_v = bufs[b][1]
            osem = bufs[b][2]
            return pltpu.make_async_copy(
                rows_v,
                out.at[pl.ds((tile_row + g * GATHER_GROUP) * 128, nrows)],
                osem)

        idx_dma(0, 0).start()
        idx_dma(1, 1).start()

        def group(g, _):
            b = lax.rem(g, 2)

            def body(bi):
                idx_v, rows_v, _, _ = bufs[bi]
                @pl.when(g >= 2)
                def _():
                    out_dma(g, bi).wait()
                idx_dma(g, bi).wait()
                descs = [
                    pltpu.make_async_copy(
                        ab_sp.at[idx_v.at[j]],
                        rows_v.at[pl.ds(j * 128, 128)],
                        gsem,
                    )
                    for j in range(GATHER_GROUP)
                ]
                for d_ in descs:
                    d_.start()
                for d_ in descs:
                    d_.wait()
                out_dma(g, bi).start()
                @pl.when(g + 2 < G_GROUPS)
                def _():
                    idx_dma(g + 2, bi).start()

            @pl.when(b == 0)
            def _():
                body(0)

            @pl.when(b == 1)
            def _():
                body(1)

            return 0

        lax.fori_loop(0, G_GROUPS, group, 0)
        out_dma(G_GROUPS - 2, 0).wait()
        out_dma(G_GROUPS - 1, 1).wait()

    @pl.when(c == 0)
    def _():
        run(src_hbm, pre_s_hbm)

    @pl.when(c == 1)
    def _():
        run(dst_hbm, pre_d_hbm)


def _gather(ab, src2d, dst2d):
    return pl.kernel(
        _gather_sc_body,
        out_type=[
            jax.ShapeDtypeStruct((E_PAD, D), jnp.float32),
            jax.ShapeDtypeStruct((E_PAD, D), jnp.float32),
        ],
        mesh=plsc.VectorSubcoreMesh(core_axis_name="c", subcore_axis_name="s"),
        scratch_types=[
            pltpu.VMEM((GATHER_GROUP, 128), jnp.int32),
            pltpu.VMEM((GATHER_GROUP, 128), jnp.int32),
            pltpu.VMEM((GATHER_GROUP * 128, D), jnp.float32),
            pltpu.VMEM((GATHER_GROUP * 128, D), jnp.float32),
            pltpu.VMEM_SHARED((N_PAD, D), jnp.float32),
            pltpu.SemaphoreType.DMA,
            pltpu.SemaphoreType.DMA,
            pltpu.SemaphoreType.DMA,
            pltpu.SemaphoreType.DMA,
            pltpu.SemaphoreType.DMA,
        ],
    )(ab, src2d, dst2d)


def _mlp_body(pre_s_ref, pre_d_ref, bin_ref, wh_ref, bh_ref, wout_ref,
              bout_ref, m_ref):
    pre = (pre_s_ref[:, :HID].astype(jnp.float32)
           + pre_d_ref[:, HID:].astype(jnp.float32))
    h = jax.nn.relu(pre + bin_ref[...])
    h = jax.nn.relu(
        jnp.dot(h, wh_ref[...], preferred_element_type=jnp.float32)
        + bh_ref[...])
    m_ref[...] = jax.nn.relu(
        jnp.dot(h, wout_ref[...], preferred_element_type=jnp.float32)
        + bout_ref[...])


def _mlp(pre_s, pre_d, bin_s, wh, bh, wout, bout):
    grid = E_PAD // MLP_BLK
    return pl.pallas_call(
        _mlp_body,
        grid=(grid,),
        in_specs=[
            pl.BlockSpec((MLP_BLK, D), lambda i: (i, 0)),
            pl.BlockSpec((MLP_BLK, D), lambda i: (i, 0)),
            pl.BlockSpec((1, HID), lambda i: (0, 0)),
            pl.BlockSpec((HID, HID), lambda i: (0, 0)),
            pl.BlockSpec((1, HID), lambda i: (0, 0)),
            pl.BlockSpec((HID, D), lambda i: (0, 0)),
            pl.BlockSpec((1, D), lambda i: (0, 0)),
        ],
        out_specs=pl.BlockSpec((MLP_BLK, D), lambda i: (i, 0)),
        out_shape=jax.ShapeDtypeStruct((E_PAD, D), jnp.float32),
    )(pre_s, pre_d, bin_s, wh, bh, wout, bout)


SC_ROWS_PER_TILE = IDX_ROWS // 32
def _scatter_sc_body(m_hbm, dst_hbm, zeros_hbm, p0_hbm, p1_hbm,
                     idx0_v, idx1_v, rows0_v, rows1_v, acc,
                     isem0, isem1, msem0, msem1):
    c = lax.axis_index("c")
    s = lax.axis_index("s")
    bufs = [(idx0_v, rows0_v, isem0, msem0), (idx1_v, rows1_v, isem1, msem1)]

    def run(out):
        tile_row = (c * 16 + s) * SC_ROWS_PER_TILE

        def idx_dma(g, b):
            return pltpu.make_async_copy(
                dst_hbm.at[pl.ds(tile_row + g, 1)], bufs[b][0], bufs[b][2])

        def m_dma(g, b):
            return pltpu.make_async_copy(
                m_hbm.at[pl.ds((tile_row + g) * 128, 128)],
                bufs[b][1], bufs[b][3])

        idx_dma(0, 0).start()
        m_dma(0, 0).start()
        idx_dma(1, 1).start()
        m_dma(1, 1).start()
        pltpu.sync_copy(zeros_hbm, acc.at[pl.ds(s * 640, 640)])
        plsc.subcore_barrier()

        def group(g, _):
            b = lax.rem(g, 2)

            def body(bi):
                idx_v, rows_v, _, _ = bufs[bi]
                idx_dma(g, bi).wait()
                m_dma(g, bi).wait()
                pltpu.sync_copy(rows_v, acc.at[idx_v.at[0]], add=True)
                @pl.when(g + 2 < SC_ROWS_PER_TILE)
                def _():
                    idx_dma(g + 2, bi).start()
                    m_dma(g + 2, bi).start()

            @pl.when(b == 0)
            def _():
                body(0)

            @pl.when(b == 1)
            def _():
                body(1)

            return 0

        lax.fori_loop(0, SC_ROWS_PER_TILE, group, 0)
        plsc.subcore_barrier()
        pltpu.sync_copy(acc.at[pl.ds(s * 640, 640)],
                        out.at[pl.ds(s * 640, 640)])

    @pl.when(c == 0)
    def _():
        run(p0_hbm)

    @pl.when(c == 1)
    def _():
        run(p1_hbm)


def _scatter(m, dst2d, zeros_rows):
    return pl.kernel(
        _scatter_sc_body,
        out_type=[
            jax.ShapeDtypeStruct((N_PAD, D), jnp.float32),
            jax.ShapeDtypeStruct((N_PAD, D), jnp.float32),
        ],
        mesh=plsc.VectorSubcoreMesh(core_axis_name="c", subcore_axis_name="s"),
        scratch_types=[
            pltpu.VMEM((1, 128), jnp.int32),
            pltpu.VMEM((1, 128), jnp.int32),
            pltpu.VMEM((128, D), jnp.float32),
            pltpu.VMEM((128, D), jnp.float32),
            pltpu.VMEM_SHARED((N_PAD, D), jnp.float32),
            pltpu.SemaphoreType.DMA,
            pltpu.SemaphoreType.DMA,
            pltpu.SemaphoreType.DMA,
            pltpu.SemaphoreType.DMA,
        ],
    )(m, dst2d, zeros_rows)


def _readout_body(p0_ref, p1_ref, ids_ref, w1_ref, b1_ref, w2_ref, b2_ref,
                  wo_ref, bo_ref, out_ref):
    gids = lax.broadcasted_iota(jnp.int32, (G, N_PAD), 0).astype(jnp.float32)
    oh = jnp.where(ids_ref[...] == gids, 1.0, 0.0)
    state = p0_ref[...] + p1_ref[...]
    mol = jnp.dot(oh, state, preferred_element_type=jnp.float32)
    h = jax.nn.relu(
        jnp.dot(mol, w1_ref[...], preferred_element_type=jnp.float32)
        + b1_ref[...])
    h = jax.nn.relu(
        jnp.dot(h, w2_ref[...], preferred_element_type=jnp.float32)
        + b2_ref[...])
    out_ref[...] = (
        jnp.dot(h, wo_ref[...], preferred_element_type=jnp.float32)
        + bo_ref[...])


def _readout(p0, p1, ids_f, w1, b1, w2, b2, wo, bo):
    return pl.pallas_call(
        _readout_body,
        out_shape=jax.ShapeDtypeStruct((G, OUT), jnp.float32),
    )(p0, p1, ids_f, w1, b1, w2, b2, wo, bo)


def kernel(x, edge_index, graph_ids, Win, bin_, Wh, bh, Wout, bout,
           W1, b1, W2, b2, Wo, bo):
    src = edge_index[0]
    dst = edge_index[1]
    pad_e = E_PAD - E_EDGES
    src2d = jnp.concatenate(
        [src, jnp.zeros((pad_e,), jnp.int32)]).reshape(IDX_ROWS, 128)
    dst2d = jnp.concatenate(
        [dst, jnp.full((pad_e,), SINK, jnp.int32)]).reshape(IDX_ROWS, 128)

    xp = jnp.pad(x, ((0, N_PAD - N_NODES), (0, 0)))
    ids_f = jnp.concatenate(
        [graph_ids, jnp.full((N_PAD - N_NODES,), G, jnp.int32)]
    ).astype(jnp.float32).reshape(1, N_PAD)
    zeros_rows = jnp.zeros((640, D), jnp.float32)
    zeros_n = jnp.zeros((N_PAD, D), jnp.float32)

    p0, p1 = xp, zeros_n
    for step in range(3):
        wcat = jnp.concatenate([Win[step][:D], Win[step][D:]], axis=1)
        ab = _proj(p0, p1, wcat)
        pre_s, pre_d = _gather(ab, src2d, dst2d)
        m = _mlp(pre_s, pre_d, bin_[step].reshape(1, HID), Wh[step],
                 bh[step].reshape(1, HID), Wout[step],
                 bout[step].reshape(1, D))
        p0, p1 = _scatter(m, dst2d, zeros_rows)

    return _readout(p0, p1, ids_f, W1, b1.reshape(1, HID), W2,
                    b2.reshape(1, HID), Wo, bo.reshape(1, OUT))
